# Initial kernel scaffold; baseline (speedup 1.0000x reference)
import jax
import jax.numpy as jnp
from jax.experimental import pallas as pl


def kernel(h, coords, edge_index, edge_type, emb, W1, b1, W2, b2, U1, ub1, U2, ub2, ln_g, ln_b):
    raise NotImplementedError("write your pallas kernel here")


# speedup vs baseline: 3.1812x; 3.1812x over previous
import jax
import jax.numpy as jnp
from jax import lax
from jax.experimental import pallas as pl
from jax.experimental.pallas import tpu as pltpu
from jax.experimental.pallas import tpu_sc as plsc

N = 10000
E = 320000
D = 128
NUM_RBF = 16
CUTOFF = 4.0
NTYPES = 8

NC = 2
NS = 16
NW = NC * NS
EPW = E // NW
CB = 80
NCH = EPW // CB
NROWS = N // NS

_STEP = CUTOFF / (NUM_RBF - 1)
_GAMMA = 1.0 / (_STEP * _STEP)

_MESH = plsc.VectorSubcoreMesh(core_axis_name="c", subcore_axis_name="s")
_SC_PARAMS = pltpu.CompilerParams(use_tc_tiling_on_sc=False)


def _precompute_body(h_ref, wa_ref, wb_ref, oa_ref, ob_ref):
    hh = h_ref[...]
    oa_ref[...] = jnp.dot(hh, wa_ref[...], preferred_element_type=jnp.float32)
    ob_ref[...] = jnp.dot(hh, wb_ref[...], preferred_element_type=jnp.float32)


def _precompute(h, w1a, w1b):
    bn = 400
    grid = (N // bn,)
    return pl.pallas_call(
        _precompute_body,
        grid=grid,
        in_specs=[
            pl.BlockSpec((bn, D), lambda i: (i, 0)),
            pl.BlockSpec((D, D), lambda i: (0, 0)),
            pl.BlockSpec((D, D), lambda i: (0, 0)),
        ],
        out_specs=[
            pl.BlockSpec((bn, D), lambda i: (i, 0)),
            pl.BlockSpec((bn, D), lambda i: (i, 0)),
        ],
        out_shape=[
            jax.ShapeDtypeStruct((N, D), jnp.float32),
            jax.ShapeDtypeStruct((N, D), jnp.float32),
        ],
    )(h, w1a, w1b)


DW = D + 16


def _sc_gather_body(ts, td, src2, dst2, zeros_c, ones_c,
                    hs_g, hd_g, cnt_out,
                    sidx, didx, hbuf, hbuf2, ones_v, cnt_sh,
                    sem1, sem2):
    c = lax.axis_index("c")
    s = lax.axis_index("s")
    wid = c * NS + s

    @pl.when(s == 0)
    def _():
        pltpu.sync_copy(zeros_c, cnt_sh)

    pltpu.sync_copy(ones_c, ones_v)
    row0 = wid * NCH
    pltpu.sync_copy(src2.at[pl.ds(row0, NCH)], sidx)
    pltpu.sync_copy(dst2.at[pl.ds(row0, NCH)], didx)
    plsc.subcore_barrier()

    def step(j, carry):
        e0 = wid * EPW + j * CB
        si = sidx.at[j]
        di = didx.at[j]
        cp1 = pltpu.async_copy(ts.at[si], hbuf, sem1)
        cp2 = pltpu.async_copy(td.at[di], hbuf2, sem2)
        cp1.wait()
        cp2.wait()
        pltpu.sync_copy(hbuf, hs_g.at[pl.ds(e0, CB)])
        pltpu.sync_copy(hbuf2, hd_g.at[pl.ds(e0, CB)])
        pltpu.sync_copy(ones_v, cnt_sh.at[di], add=True)
        return carry

    lax.fori_loop(0, NCH, step, 0)
    plsc.subcore_barrier()

    @pl.when(s == 0)
    def _():
        pltpu.sync_copy(cnt_sh, cnt_out.at[c])


def _sc_gather(ts, td, src2, dst2):
    zeros_c = jnp.zeros((N, 16), jnp.float32)
    ones_c = jnp.ones((CB, 16), jnp.float32)
    out_type = [
        jax.ShapeDtypeStruct((E, DW), jnp.float32),
        jax.ShapeDtypeStruct((E, DW), jnp.float32),
        jax.ShapeDtypeStruct((NC, N, 16), jnp.float32),
    ]
    scratch = [
        pltpu.VMEM((NCH, CB), jnp.int32),
        pltpu.VMEM((NCH, CB), jnp.int32),
        pltpu.VMEM((CB, DW), jnp.float32),
        pltpu.VMEM((CB, DW), jnp.float32),
        pltpu.VMEM((CB, 16), jnp.float32),
        pltpu.VMEM_SHARED((N, 16), jnp.float32),
        pltpu.SemaphoreType.DMA,
        pltpu.SemaphoreType.DMA,
    ]
    return pl.kernel(
        _sc_gather_body,
        out_type=out_type,
        mesh=_MESH,
        scratch_types=scratch,
        compiler_params=_SC_PARAMS,
    )(ts, td, src2, dst2, zeros_c, ones_c)


def _edge_body(hs_ref, hd_ref, et_ref,
               emb_ref, w1e_ref, b1_ref, w1r_ref, w1d_ref, w2_ref, b2_ref,
               out_ref):
    rel = hs_ref[:, D:DW] - hd_ref[:, D:DW]
    d2 = jnp.sum(rel * rel, axis=1, keepdims=True)
    dist = jnp.sqrt(d2)
    centers = lax.broadcasted_iota(
        jnp.int32, (1, NUM_RBF), 1).astype(jnp.float32) * _STEP
    rad = jnp.exp(-_GAMMA * jnp.square(dist - centers))
    embw = jnp.dot(emb_ref[...], w1e_ref[...],
                   preferred_element_type=jnp.float32) + b1_ref[...]
    types = lax.broadcasted_iota(jnp.int32, (1, NTYPES), 1)
    oh = (et_ref[...] == types).astype(jnp.float32)
    z = (hs_ref[:, :D] + hd_ref[:, :D]
         + jnp.dot(rad, w1r_ref[...], preferred_element_type=jnp.float32)
         + dist * w1d_ref[...]
         + jnp.dot(oh, embw, preferred_element_type=jnp.float32))
    m1 = z * jax.nn.sigmoid(z)
    m2 = jnp.dot(m1, w2_ref[...], preferred_element_type=jnp.float32) + b2_ref[...]
    out_ref[...] = m2 * jax.nn.sigmoid(m2)


def _edge_mlp(hs_g, hd_g, et2, emb, w1e, b1r, w1r, w1dr, w2, b2r):
    be = 512
    grid = (E // be,)
    return pl.pallas_call(
        _edge_body,
        grid=grid,
        in_specs=[
            pl.BlockSpec((be, DW), lambda i: (i, 0)),
            pl.BlockSpec((be, DW), lambda i: (i, 0)),
            pl.BlockSpec((be, 1), lambda i: (i, 0)),
            pl.BlockSpec((NTYPES, D), lambda i: (0, 0)),
            pl.BlockSpec((D, D), lambda i: (0, 0)),
            pl.BlockSpec((1, D), lambda i: (0, 0)),
            pl.BlockSpec((NUM_RBF, D), lambda i: (0, 0)),
            pl.BlockSpec((1, D), lambda i: (0, 0)),
            pl.BlockSpec((D, D), lambda i: (0, 0)),
            pl.BlockSpec((1, D), lambda i: (0, 0)),
        ],
        out_specs=pl.BlockSpec((be, D), lambda i: (i, 0)),
        out_shape=jax.ShapeDtypeStruct((E, D), jnp.float32),
    )(hs_g, hd_g, et2, emb, w1e, b1r, w1r, w1dr, w2, b2r)


def _sc_scatter_body(m, dst2, zeros128, sums_out, didx, mbuf, sums_sh):
    c = lax.axis_index("c")
    s = lax.axis_index("s")
    wid = c * NS + s

    pltpu.sync_copy(zeros128.at[pl.ds(s * NROWS, NROWS)],
                    sums_sh.at[pl.ds(s * NROWS, NROWS)])
    row0 = wid * NCH
    pltpu.sync_copy(dst2.at[pl.ds(row0, NCH)], didx)
    plsc.subcore_barrier()

    def step(j, carry):
        e0 = wid * EPW + j * CB
        pltpu.sync_copy(m.at[pl.ds(e0, CB)], mbuf)
        pltpu.sync_copy(mbuf, sums_sh.at[didx.at[j]], add=True)
        return carry

    lax.fori_loop(0, NCH, step, 0)
    plsc.subcore_barrier()
    pltpu.sync_copy(sums_sh.at[pl.ds(s * NROWS, NROWS)],
                    sums_out.at[c, pl.ds(s * NROWS, NROWS)])


def _sc_scatter(m, dst2):
    zeros128 = jnp.zeros((N, D), jnp.float32)
    scratch = [
        pltpu.VMEM((NCH, CB), jnp.int32),
        pltpu.VMEM((CB, D), jnp.float32),
        pltpu.VMEM_SHARED((N, D), jnp.float32),
    ]
    return pl.kernel(
        _sc_scatter_body,
        out_type=jax.ShapeDtypeStruct((NC, N, D), jnp.float32),
        mesh=_MESH,
        scratch_types=scratch,
        compiler_params=_SC_PARAMS,
    )(m, dst2, zeros128)


def _node_body(h_ref, s0_ref, s1_ref, c0_ref, c1_ref,
               u1a_ref, u1b_ref, ub1_ref, u2_ref, ub2_ref, g_ref, b_ref,
               out_ref):
    hh = h_ref[...]
    cnt = jnp.maximum(c0_ref[:, :1] + c1_ref[:, :1], 1.0)
    agg = (s0_ref[...] + s1_ref[...]) / cnt
    t = (jnp.dot(hh, u1a_ref[...], preferred_element_type=jnp.float32)
         + jnp.dot(agg, u1b_ref[...], preferred_element_type=jnp.float32)
         + ub1_ref[...])
    t = t * jax.nn.sigmoid(t)
    u = jnp.dot(t, u2_ref[...], preferred_element_type=jnp.float32) + ub2_ref[...]
    x = hh + u
    mu = jnp.mean(x, axis=-1, keepdims=True)
    var = jnp.mean(jnp.square(x - mu), axis=-1, keepdims=True)
    out_ref[...] = (x - mu) / jnp.sqrt(var + 1e-5) * g_ref[...] + b_ref[...]


def _node_update(h, s0, s1, c0, c1, u1a, u1b, ub1r, u2, ub2r, gr, br):
    bn = 400
    grid = (N // bn,)
    return pl.pallas_call(
        _node_body,
        grid=grid,
        in_specs=[
            pl.BlockSpec((bn, D), lambda i: (i, 0)),
            pl.BlockSpec((bn, D), lambda i: (i, 0)),
            pl.BlockSpec((bn, D), lambda i: (i, 0)),
            pl.BlockSpec((bn, 16), lambda i: (i, 0)),
            pl.BlockSpec((bn, 16), lambda i: (i, 0)),
            pl.BlockSpec((D, D), lambda i: (0, 0)),
            pl.BlockSpec((D, D), lambda i: (0, 0)),
            pl.BlockSpec((1, D), lambda i: (0, 0)),
            pl.BlockSpec((D, D), lambda i: (0, 0)),
            pl.BlockSpec((1, D), lambda i: (0, 0)),
            pl.BlockSpec((1, D), lambda i: (0, 0)),
            pl.BlockSpec((1, D), lambda i: (0, 0)),
        ],
        out_specs=pl.BlockSpec((bn, D), lambda i: (i, 0)),
        out_shape=jax.ShapeDtypeStruct((N, D), jnp.float32),
    )(h, s0, s1, c0, c1, u1a, u1b, ub1r, u2, ub2r, gr, br)


def kernel(h, coords, edge_index, edge_type, emb, W1, b1, W2, b2,
           U1, ub1, U2, ub2, ln_g, ln_b):
    src = edge_index[0].astype(jnp.int32)
    dst = edge_index[1].astype(jnp.int32)
    src2 = src.reshape(E // CB, CB)
    dst2 = dst.reshape(E // CB, CB)
    et2 = edge_type.astype(jnp.int32).reshape(E, 1)
    coordsp = jnp.pad(coords.astype(jnp.float32), ((0, 0), (0, 1)))

    w1a = W1[:D]
    w1b = W1[D:2 * D]
    w1e = W1[2 * D:3 * D]
    w1r = W1[3 * D:3 * D + NUM_RBF]
    w1dr = W1[3 * D + NUM_RBF:]
    b1r = b1.reshape(1, D)
    b2r = b2.reshape(1, D)
    u1a = U1[:D]
    u1b = U1[D:]
    ub1r = ub1.reshape(1, D)
    ub2r = ub2.reshape(1, D)
    gr = ln_g.reshape(1, D)
    br = ln_b.reshape(1, D)

    hws, hwd = _precompute(h, w1a, w1b)
    zpad = jnp.zeros((N, DW - D - 4), jnp.float32)
    ts = jnp.concatenate([hws, coordsp, zpad], axis=1)
    td = jnp.concatenate([hwd, coordsp, zpad], axis=1)
    hs_g, hd_g, cnt = _sc_gather(ts, td, src2, dst2)
    m = _edge_mlp(hs_g, hd_g, et2, emb, w1e, b1r, w1r, w1dr, W2, b2r)
    sums = _sc_scatter(m, dst2)
    return _node_update(h, sums[0], sums[1], cnt[0], cnt[1],
                        u1a, u1b, ub1r, U2, ub2r, gr, br)


# speedup vs baseline: 4.4253x; 1.3911x over previous
import jax
import jax.numpy as jnp
from jax import lax
from jax.experimental import pallas as pl
from jax.experimental.pallas import tpu as pltpu
from jax.experimental.pallas import tpu_sc as plsc

N = 10000
E = 320000
D = 128
NUM_RBF = 16
CUTOFF = 4.0
NTYPES = 8

NC = 2
NS = 16
NW = NC * NS
EPW = E // NW
CB = 80
NCH = EPW // CB
NROWS = N // NS

_STEP = CUTOFF / (NUM_RBF - 1)
_GAMMA = 1.0 / (_STEP * _STEP)

_MESH = plsc.VectorSubcoreMesh(core_axis_name="c", subcore_axis_name="s")
_SC_PARAMS = pltpu.CompilerParams(use_tc_tiling_on_sc=False)


def _precompute_body(h_ref, wa_ref, wb_ref, oa_ref, ob_ref):
    hh = h_ref[...]
    oa_ref[...] = jnp.dot(hh, wa_ref[...], preferred_element_type=jnp.float32)
    ob_ref[...] = jnp.dot(hh, wb_ref[...], preferred_element_type=jnp.float32)


def _precompute(h, w1a, w1b):
    bn = 400
    grid = (N // bn,)
    return pl.pallas_call(
        _precompute_body,
        grid=grid,
        in_specs=[
            pl.BlockSpec((bn, D), lambda i: (i, 0)),
            pl.BlockSpec((D, D), lambda i: (0, 0)),
            pl.BlockSpec((D, D), lambda i: (0, 0)),
        ],
        out_specs=[
            pl.BlockSpec((bn, D), lambda i: (i, 0)),
            pl.BlockSpec((bn, D), lambda i: (i, 0)),
        ],
        out_shape=[
            jax.ShapeDtypeStruct((N, D), jnp.float32),
            jax.ShapeDtypeStruct((N, D), jnp.float32),
        ],
    )(h, w1a, w1b)


DW = D + 16


def _sc_gather_body(ts, td, c16, c16n, src2, dst2, zeros_c, ones_c,
                    hsum, rel16, cnt_out,
                    sidx, didx, hbuf, cbuf, ones_v, cnt_sh,
                    sem1, sem2):
    c = lax.axis_index("c")
    s = lax.axis_index("s")
    wid = c * NS + s

    @pl.when(s == 0)
    def _():
        pltpu.sync_copy(zeros_c, cnt_sh)

    pltpu.sync_copy(ones_c, ones_v)
    row0 = wid * NCH
    pltpu.sync_copy(src2.at[pl.ds(row0, NCH)], sidx)
    pltpu.sync_copy(dst2.at[pl.ds(row0, NCH)], didx)
    plsc.subcore_barrier()

    def step(j, carry):
        e0 = wid * EPW + j * CB
        si = sidx.at[j]
        di = didx.at[j]
        cp1 = pltpu.async_copy(ts.at[si], hbuf, sem1)
        cp2 = pltpu.async_copy(c16.at[si], cbuf, sem2)
        cp1.wait()
        cp2.wait()
        cp3 = pltpu.async_copy(td.at[di], hbuf, sem1, add=True)
        cp4 = pltpu.async_copy(c16n.at[di], cbuf, sem2, add=True)
        cp3.wait()
        cp4.wait()
        pltpu.sync_copy(hbuf, hsum.at[pl.ds(e0, CB)])
        pltpu.sync_copy(cbuf, rel16.at[pl.ds(e0, CB)])
        pltpu.sync_copy(ones_v, cnt_sh.at[di], add=True)
        return carry

    lax.fori_loop(0, NCH, step, 0)
    plsc.subcore_barrier()

    @pl.when(s == 0)
    def _():
        pltpu.sync_copy(cnt_sh, cnt_out.at[c])


def _sc_gather(ts, td, c16, c16n, src2, dst2):
    zeros_c = jnp.zeros((N, 16), jnp.float32)
    ones_c = jnp.ones((CB, 16), jnp.float32)
    out_type = [
        jax.ShapeDtypeStruct((E, D), jnp.float32),
        jax.ShapeDtypeStruct((E, 16), jnp.float32),
        jax.ShapeDtypeStruct((NC, N, 16), jnp.float32),
    ]
    scratch = [
        pltpu.VMEM((NCH, CB), jnp.int32),
        pltpu.VMEM((NCH, CB), jnp.int32),
        pltpu.VMEM((CB, D), jnp.float32),
        pltpu.VMEM((CB, 16), jnp.float32),
        pltpu.VMEM((CB, 16), jnp.float32),
        pltpu.VMEM_SHARED((N, 16), jnp.float32),
        pltpu.SemaphoreType.DMA,
        pltpu.SemaphoreType.DMA,
    ]
    return pl.kernel(
        _sc_gather_body,
        out_type=out_type,
        mesh=_MESH,
        scratch_types=scratch,
        compiler_params=_SC_PARAMS,
    )(ts, td, c16, c16n, src2, dst2, zeros_c, ones_c)


def _edge_body(hs_ref, rel_ref, et_ref,
               emb_ref, w1e_ref, b1_ref, w1r_ref, w1d_ref, w2_ref, b2_ref,
               out_ref):
    rel = rel_ref[...]
    d2 = jnp.sum(rel * rel, axis=1, keepdims=True)
    dist = jnp.sqrt(d2)
    centers = lax.broadcasted_iota(
        jnp.int32, (1, NUM_RBF), 1).astype(jnp.float32) * _STEP
    rad = jnp.exp(-_GAMMA * jnp.square(dist - centers))
    embw = jnp.dot(emb_ref[...], w1e_ref[...],
                   preferred_element_type=jnp.float32) + b1_ref[...]
    types = lax.broadcasted_iota(jnp.int32, (1, NTYPES), 1)
    oh = (et_ref[...] == types).astype(jnp.float32)
    z = (hs_ref[...]
         + jnp.dot(rad, w1r_ref[...], preferred_element_type=jnp.float32)
         + dist * w1d_ref[...]
         + jnp.dot(oh, embw, preferred_element_type=jnp.float32))
    m1 = z * jax.nn.sigmoid(z)
    m2 = jnp.dot(m1, w2_ref[...], preferred_element_type=jnp.float32) + b2_ref[...]
    out_ref[...] = m2 * jax.nn.sigmoid(m2)


def _edge_mlp(hsum, rel16, et2, emb, w1e, b1r, w1r, w1dr, w2, b2r):
    be = 512
    grid = (E // be,)
    return pl.pallas_call(
        _edge_body,
        grid=grid,
        in_specs=[
            pl.BlockSpec((be, D), lambda i: (i, 0)),
            pl.BlockSpec((be, 16), lambda i: (i, 0)),
            pl.BlockSpec((be, 1), lambda i: (i, 0)),
            pl.BlockSpec((NTYPES, D), lambda i: (0, 0)),
            pl.BlockSpec((D, D), lambda i: (0, 0)),
            pl.BlockSpec((1, D), lambda i: (0, 0)),
            pl.BlockSpec((NUM_RBF, D), lambda i: (0, 0)),
            pl.BlockSpec((1, D), lambda i: (0, 0)),
            pl.BlockSpec((D, D), lambda i: (0, 0)),
            pl.BlockSpec((1, D), lambda i: (0, 0)),
        ],
        out_specs=pl.BlockSpec((be, D), lambda i: (i, 0)),
        out_shape=jax.ShapeDtypeStruct((E, D), jnp.float32),
    )(hsum, rel16, et2, emb, w1e, b1r, w1r, w1dr, w2, b2r)


def _sc_scatter_body(m, dst2, zeros128, sums_out, didx, mbuf, sums_sh):
    c = lax.axis_index("c")
    s = lax.axis_index("s")
    wid = c * NS + s

    pltpu.sync_copy(zeros128.at[pl.ds(s * NROWS, NROWS)],
                    sums_sh.at[pl.ds(s * NROWS, NROWS)])
    row0 = wid * NCH
    pltpu.sync_copy(dst2.at[pl.ds(row0, NCH)], didx)
    plsc.subcore_barrier()

    def step(j, carry):
        e0 = wid * EPW + j * CB
        pltpu.sync_copy(m.at[pl.ds(e0, CB)], mbuf)
        pltpu.sync_copy(mbuf, sums_sh.at[didx.at[j]], add=True)
        return carry

    lax.fori_loop(0, NCH, step, 0)
    plsc.subcore_barrier()
    pltpu.sync_copy(sums_sh.at[pl.ds(s * NROWS, NROWS)],
                    sums_out.at[c, pl.ds(s * NROWS, NROWS)])


def _sc_scatter(m, dst2):
    zeros128 = jnp.zeros((N, D), jnp.float32)
    scratch = [
        pltpu.VMEM((NCH, CB), jnp.int32),
        pltpu.VMEM((CB, D), jnp.float32),
        pltpu.VMEM_SHARED((N, D), jnp.float32),
    ]
    return pl.kernel(
        _sc_scatter_body,
        out_type=jax.ShapeDtypeStruct((NC, N, D), jnp.float32),
        mesh=_MESH,
        scratch_types=scratch,
        compiler_params=_SC_PARAMS,
    )(m, dst2, zeros128)


def _node_body(h_ref, s0_ref, s1_ref, c0_ref, c1_ref,
               u1a_ref, u1b_ref, ub1_ref, u2_ref, ub2_ref, g_ref, b_ref,
               out_ref):
    hh = h_ref[...]
    cnt = jnp.maximum(c0_ref[:, :1] + c1_ref[:, :1], 1.0)
    agg = (s0_ref[...] + s1_ref[...]) / cnt
    t = (jnp.dot(hh, u1a_ref[...], preferred_element_type=jnp.float32)
         + jnp.dot(agg, u1b_ref[...], preferred_element_type=jnp.float32)
         + ub1_ref[...])
    t = t * jax.nn.sigmoid(t)
    u = jnp.dot(t, u2_ref[...], preferred_element_type=jnp.float32) + ub2_ref[...]
    x = hh + u
    mu = jnp.mean(x, axis=-1, keepdims=True)
    var = jnp.mean(jnp.square(x - mu), axis=-1, keepdims=True)
    out_ref[...] = (x - mu) / jnp.sqrt(var + 1e-5) * g_ref[...] + b_ref[...]


def _node_update(h, s0, s1, c0, c1, u1a, u1b, ub1r, u2, ub2r, gr, br):
    bn = 400
    grid = (N // bn,)
    return pl.pallas_call(
        _node_body,
        grid=grid,
        in_specs=[
            pl.BlockSpec((bn, D), lambda i: (i, 0)),
            pl.BlockSpec((bn, D), lambda i: (i, 0)),
            pl.BlockSpec((bn, D), lambda i: (i, 0)),
            pl.BlockSpec((bn, 16), lambda i: (i, 0)),
            pl.BlockSpec((bn, 16), lambda i: (i, 0)),
            pl.BlockSpec((D, D), lambda i: (0, 0)),
            pl.BlockSpec((D, D), lambda i: (0, 0)),
            pl.BlockSpec((1, D), lambda i: (0, 0)),
            pl.BlockSpec((D, D), lambda i: (0, 0)),
            pl.BlockSpec((1, D), lambda i: (0, 0)),
            pl.BlockSpec((1, D), lambda i: (0, 0)),
            pl.BlockSpec((1, D), lambda i: (0, 0)),
        ],
        out_specs=pl.BlockSpec((bn, D), lambda i: (i, 0)),
        out_shape=jax.ShapeDtypeStruct((N, D), jnp.float32),
    )(h, s0, s1, c0, c1, u1a, u1b, ub1r, u2, ub2r, gr, br)


def kernel(h, coords, edge_index, edge_type, emb, W1, b1, W2, b2,
           U1, ub1, U2, ub2, ln_g, ln_b):
    src = edge_index[0].astype(jnp.int32)
    dst = edge_index[1].astype(jnp.int32)
    src2 = src.reshape(E // CB, CB)
    dst2 = dst.reshape(E // CB, CB)
    et2 = edge_type.astype(jnp.int32).reshape(E, 1)
    coordsp = jnp.pad(coords.astype(jnp.float32), ((0, 0), (0, 1)))

    w1a = W1[:D]
    w1b = W1[D:2 * D]
    w1e = W1[2 * D:3 * D]
    w1r = W1[3 * D:3 * D + NUM_RBF]
    w1dr = W1[3 * D + NUM_RBF:]
    b1r = b1.reshape(1, D)
    b2r = b2.reshape(1, D)
    u1a = U1[:D]
    u1b = U1[D:]
    ub1r = ub1.reshape(1, D)
    ub2r = ub2.reshape(1, D)
    gr = ln_g.reshape(1, D)
    br = ln_b.reshape(1, D)

    hws, hwd = _precompute(h, w1a, w1b)
    cpad = jnp.zeros((N, 13), jnp.float32)
    c16 = jnp.concatenate([coords.astype(jnp.float32), cpad], axis=1)
    c16n = -c16
    hsum, rel16, cnt = _sc_gather(hws, hwd, c16, c16n, src2, dst2)
    m = _edge_mlp(hsum, rel16, et2, emb, w1e, b1r, w1r, w1dr, W2, b2r)
    sums = _sc_scatter(m, dst2)
    return _node_update(h, sums[0], sums[1], cnt[0], cnt[1],
                        u1a, u1b, ub1r, U2, ub2r, gr, br)


# speedup vs baseline: 6.1320x; 1.3857x over previous
import jax
import jax.numpy as jnp
from jax import lax
from jax.experimental import pallas as pl
from jax.experimental.pallas import tpu as pltpu
from jax.experimental.pallas import tpu_sc as plsc

N = 10000
E = 320000
D = 128
NUM_RBF = 16
CUTOFF = 4.0
NTYPES = 8

NC = 2
NS = 16
NW = NC * NS
CB = 80

SLICES = 5
ESL = E // SLICES
EPW = ESL // NW
NCH = EPW // CB
ROWS_SL = ESL // CB
NROWS = N // NS

_STEP = CUTOFF / (NUM_RBF - 1)
_GAMMA = 1.0 / (_STEP * _STEP)

_MESH = plsc.VectorSubcoreMesh(core_axis_name="c", subcore_axis_name="s")
_SC_PARAMS = pltpu.CompilerParams(use_tc_tiling_on_sc=False)


def _precompute_body(h_ref, wa_ref, wb_ref, oa_ref, ob_ref):
    hh = h_ref[...]
    oa_ref[...] = jnp.dot(hh, wa_ref[...], preferred_element_type=jnp.float32)
    ob_ref[...] = jnp.dot(hh, wb_ref[...], preferred_element_type=jnp.float32)


def _precompute(h, w1a, w1b):
    bn = 400
    grid = (N // bn,)
    return pl.pallas_call(
        _precompute_body,
        grid=grid,
        in_specs=[
            pl.BlockSpec((bn, D), lambda i: (i, 0)),
            pl.BlockSpec((D, D), lambda i: (0, 0)),
            pl.BlockSpec((D, D), lambda i: (0, 0)),
        ],
        out_specs=[
            pl.BlockSpec((bn, D), lambda i: (i, 0)),
            pl.BlockSpec((bn, D), lambda i: (i, 0)),
        ],
        out_shape=[
            jax.ShapeDtypeStruct((N, D), jnp.float32),
            jax.ShapeDtypeStruct((N, D), jnp.float32),
        ],
    )(h, w1a, w1b)


def _sc_gather_body(ts, td, c16, c16n, src2, dst2,
                    hsum, rel16,
                    sidx, didx, hbuf, cbuf,
                    sem1, sem2):
    c = lax.axis_index("c")
    s = lax.axis_index("s")
    wid = c * NS + s

    row0 = wid * NCH
    pltpu.sync_copy(src2.at[pl.ds(row0, NCH)], sidx)
    pltpu.sync_copy(dst2.at[pl.ds(row0, NCH)], didx)

    def step(j, carry):
        e0 = wid * EPW + j * CB
        si = sidx.at[j]
        di = didx.at[j]
        cp1 = pltpu.async_copy(ts.at[si], hbuf, sem1)
        cp2 = pltpu.async_copy(c16.at[si], cbuf, sem2)
        cp1.wait()
        cp2.wait()
        cp3 = pltpu.async_copy(td.at[di], hbuf, sem1, add=True)
        cp4 = pltpu.async_copy(c16n.at[di], cbuf, sem2, add=True)
        cp3.wait()
        cp4.wait()
        pltpu.sync_copy(hbuf, hsum.at[pl.ds(e0, CB)])
        pltpu.sync_copy(cbuf, rel16.at[pl.ds(e0, CB)])
        return carry

    lax.fori_loop(0, NCH, step, 0)


def _sc_gather(ts, td, c16, c16n, src2_s, dst2_s):
    out_type = [
        jax.ShapeDtypeStruct((ESL, D), jnp.float32),
        jax.ShapeDtypeStruct((ESL, 16), jnp.float32),
    ]
    scratch = [
        pltpu.VMEM((NCH, CB), jnp.int32),
        pltpu.VMEM((NCH, CB), jnp.int32),
        pltpu.VMEM((CB, D), jnp.float32),
        pltpu.VMEM((CB, 16), jnp.float32),
        pltpu.SemaphoreType.DMA,
        pltpu.SemaphoreType.DMA,
    ]
    return pl.kernel(
        _sc_gather_body,
        out_type=out_type,
        mesh=_MESH,
        scratch_types=scratch,
        compiler_params=_SC_PARAMS,
    )(ts, td, c16, c16n, src2_s, dst2_s)


def _edge_body(hs_ref, rel_ref, et_ref,
               emb_ref, w1e_ref, b1_ref, w1r_ref, w1dt_ref, w2_ref, b2_ref,
               out_ref):
    rel = rel_ref[...]
    r2 = rel * rel
    ones16 = jnp.ones((16, NUM_RBF), jnp.float32)
    s = jnp.dot(r2, ones16, preferred_element_type=jnp.float32)
    dist16 = jnp.sqrt(s)
    centers = lax.broadcasted_iota(
        jnp.int32, (1, NUM_RBF), 1).astype(jnp.float32) * _STEP
    rad = jnp.exp(-_GAMMA * jnp.square(dist16 - centers))
    embw = jnp.dot(emb_ref[...], w1e_ref[...],
                   preferred_element_type=jnp.float32) + b1_ref[...]
    types = lax.broadcasted_iota(jnp.int32, (1, NTYPES), 1)
    oh = (et_ref[...] == types).astype(jnp.float32)
    z = (hs_ref[...]
         + jnp.dot(rad, w1r_ref[...], preferred_element_type=jnp.float32)
         + jnp.dot(dist16, w1dt_ref[...], preferred_element_type=jnp.float32)
         + jnp.dot(oh, embw, preferred_element_type=jnp.float32))
    m1 = z * jax.nn.sigmoid(z)
    m2 = jnp.dot(m1, w2_ref[...], preferred_element_type=jnp.float32) + b2_ref[...]
    out_ref[...] = m2 * jax.nn.sigmoid(m2)


def _edge_mlp(hsum, rel16, et2_s, emb, w1e, b1r, w1r, w1dt, w2, b2r):
    be = 1280
    grid = (ESL // be,)
    return pl.pallas_call(
        _edge_body,
        grid=grid,
        in_specs=[
            pl.BlockSpec((be, D), lambda i: (i, 0)),
            pl.BlockSpec((be, 16), lambda i: (i, 0)),
            pl.BlockSpec((be, 1), lambda i: (i, 0)),
            pl.BlockSpec((NTYPES, D), lambda i: (0, 0)),
            pl.BlockSpec((D, D), lambda i: (0, 0)),
            pl.BlockSpec((1, D), lambda i: (0, 0)),
            pl.BlockSpec((NUM_RBF, D), lambda i: (0, 0)),
            pl.BlockSpec((NUM_RBF, D), lambda i: (0, 0)),
            pl.BlockSpec((D, D), lambda i: (0, 0)),
            pl.BlockSpec((1, D), lambda i: (0, 0)),
        ],
        out_specs=pl.BlockSpec((be, D), lambda i: (i, 0)),
        out_shape=jax.ShapeDtypeStruct((ESL, D), jnp.float32),
    )(hsum, rel16, et2_s, emb, w1e, b1r, w1r, w1dt, w2, b2r)


def _sc_scatter_body(m, dst2, zeros128, zeros_c, ones_c,
                     sums_out, cnt_out,
                     didx, mbuf, ones_v, sums_sh, cnt_sh):
    c = lax.axis_index("c")
    s = lax.axis_index("s")
    wid = c * NS + s

    pltpu.sync_copy(zeros128.at[pl.ds(s * NROWS, NROWS)],
                    sums_sh.at[pl.ds(s * NROWS, NROWS)])

    @pl.when(s == 0)
    def _():
        pltpu.sync_copy(zeros_c, cnt_sh)

    pltpu.sync_copy(ones_c, ones_v)
    row0 = wid * NCH
    pltpu.sync_copy(dst2.at[pl.ds(row0, NCH)], didx)
    plsc.subcore_barrier()

    def step(j, carry):
        e0 = wid * EPW + j * CB
        di = didx.at[j]
        pltpu.sync_copy(m.at[pl.ds(e0, CB)], mbuf)
        pltpu.sync_copy(mbuf, sums_sh.at[di], add=True)
        pltpu.sync_copy(ones_v, cnt_sh.at[di], add=True)
        return carry

    lax.fori_loop(0, NCH, step, 0)
    plsc.subcore_barrier()
    pltpu.sync_copy(sums_sh.at[pl.ds(s * NROWS, NROWS)],
                    sums_out.at[c, pl.ds(s * NROWS, NROWS)])

    @pl.when(s == 0)
    def _():
        pltpu.sync_copy(cnt_sh, cnt_out.at[c])


def _sc_scatter(m_s, dst2_s):
    zeros128 = jnp.zeros((N, D), jnp.float32)
    zeros_c = jnp.zeros((N, 16), jnp.float32)
    ones_c = jnp.ones((CB, 16), jnp.float32)
    out_type = [
        jax.ShapeDtypeStruct((NC, N, D), jnp.float32),
        jax.ShapeDtypeStruct((NC, N, 16), jnp.float32),
    ]
    scratch = [
        pltpu.VMEM((NCH, CB), jnp.int32),
        pltpu.VMEM((CB, D), jnp.float32),
        pltpu.VMEM((CB, 16), jnp.float32),
        pltpu.VMEM_SHARED((N, D), jnp.float32),
        pltpu.VMEM_SHARED((N, 16), jnp.float32),
    ]
    return pl.kernel(
        _sc_scatter_body,
        out_type=out_type,
        mesh=_MESH,
        scratch_types=scratch,
        compiler_params=_SC_PARAMS,
    )(m_s, dst2_s, zeros128, zeros_c, ones_c)


def _node_body(h_ref, s0_ref, s1_ref, s2_ref, s3_ref, s4_ref,
               c0_ref, c1_ref, c2_ref, c3_ref, c4_ref,
               u1a_ref, u1b_ref, ub1_ref, u2_ref, ub2_ref, g_ref, b_ref,
               out_ref):
    hh = h_ref[...]
    cnt = (c0_ref[0, :, :1] + c0_ref[1, :, :1]
           + c1_ref[0, :, :1] + c1_ref[1, :, :1]
           + c2_ref[0, :, :1] + c2_ref[1, :, :1]
           + c3_ref[0, :, :1] + c3_ref[1, :, :1]
           + c4_ref[0, :, :1] + c4_ref[1, :, :1])
    cnt = jnp.maximum(cnt, 1.0)
    tot = (s0_ref[0] + s0_ref[1] + s1_ref[0] + s1_ref[1]
           + s2_ref[0] + s2_ref[1] + s3_ref[0] + s3_ref[1]
           + s4_ref[0] + s4_ref[1])
    agg = tot / cnt
    t = (jnp.dot(hh, u1a_ref[...], preferred_element_type=jnp.float32)
         + jnp.dot(agg, u1b_ref[...], preferred_element_type=jnp.float32)
         + ub1_ref[...])
    t = t * jax.nn.sigmoid(t)
    u = jnp.dot(t, u2_ref[...], preferred_element_type=jnp.float32) + ub2_ref[...]
    x = hh + u
    mu = jnp.mean(x, axis=-1, keepdims=True)
    var = jnp.mean(jnp.square(x - mu), axis=-1, keepdims=True)
    out_ref[...] = (x - mu) / jnp.sqrt(var + 1e-5) * g_ref[...] + b_ref[...]


def _node_update(h, sums_list, cnt_list, u1a, u1b, ub1r, u2, ub2r, gr, br):
    bn = 400
    grid = (N // bn,)
    sum_spec = pl.BlockSpec((NC, bn, D), lambda i: (0, i, 0))
    cnt_spec = pl.BlockSpec((NC, bn, 16), lambda i: (0, i, 0))
    full = lambda shape: pl.BlockSpec(shape, lambda i: (0, 0))
    return pl.pallas_call(
        _node_body,
        grid=grid,
        in_specs=[pl.BlockSpec((bn, D), lambda i: (i, 0))]
        + [sum_spec] * SLICES + [cnt_spec] * SLICES
        + [full((D, D)), full((D, D)), full((1, D)),
           full((D, D)), full((1, D)), full((1, D)), full((1, D))],
        out_specs=pl.BlockSpec((bn, D), lambda i: (i, 0)),
        out_shape=jax.ShapeDtypeStruct((N, D), jnp.float32),
    )(h, *sums_list, *cnt_list, u1a, u1b, ub1r, u2, ub2r, gr, br)


def kernel(h, coords, edge_index, edge_type, emb, W1, b1, W2, b2,
           U1, ub1, U2, ub2, ln_g, ln_b):
    src = edge_index[0].astype(jnp.int32)
    dst = edge_index[1].astype(jnp.int32)
    src2 = src.reshape(E // CB, CB)
    dst2 = dst.reshape(E // CB, CB)
    et2 = edge_type.astype(jnp.int32).reshape(E, 1)

    w1a = W1[:D]
    w1b = W1[D:2 * D]
    w1e = W1[2 * D:3 * D]
    w1r = W1[3 * D:3 * D + NUM_RBF]
    w1dr = W1[3 * D + NUM_RBF:]
    w1dt = jnp.tile(w1dr / 16.0, (16, 1))
    b1r = b1.reshape(1, D)
    b2r = b2.reshape(1, D)
    u1a = U1[:D]
    u1b = U1[D:]
    ub1r = ub1.reshape(1, D)
    ub2r = ub2.reshape(1, D)
    gr = ln_g.reshape(1, D)
    br = ln_b.reshape(1, D)

    hws, hwd = _precompute(h, w1a, w1b)
    cpad = jnp.zeros((N, 13), jnp.float32)
    c16 = jnp.concatenate([coords.astype(jnp.float32), cpad], axis=1)
    c16n = -c16

    sums_list = []
    cnt_list = []
    for sl in range(SLICES):
        src2_s = src2[sl * ROWS_SL:(sl + 1) * ROWS_SL]
        dst2_s = dst2[sl * ROWS_SL:(sl + 1) * ROWS_SL]
        et2_s = et2[sl * ESL:(sl + 1) * ESL]
        hsum, rel16 = _sc_gather(hws, hwd, c16, c16n, src2_s, dst2_s)
        m_s = _edge_mlp(hsum, rel16, et2_s, emb, w1e, b1r, w1r, w1dt, W2, b2r)
        s_s, c_s = _sc_scatter(m_s, dst2_s)
        sums_list.append(s_s)
        cnt_list.append(c_s)

    return _node_update(h, sums_list, cnt_list, u1a, u1b, ub1r, U2, ub2r, gr, br)
